# baseline (speedup 1.0000x reference)
import functools

import jax
import jax.numpy as jnp
from jax import lax
from jax.experimental import pallas as pl
from jax.experimental.pallas import tpu as pltpu
from jax.experimental.pallas import tpu_sc as plsc

N0 = 100000
N1 = 25000
N2 = 6250

NC = 2
NS = 16
GL = 512
NG = 2


def _rup(x, m):
    return (x + m - 1) // m * m


def _mesh():
    return plsc.VectorSubcoreMesh(core_axis_name="c", subcore_axis_name="s")


_SC_PARAMS = pltpu.CompilerParams(use_tc_tiling_on_sc=False)


@functools.partial(jax.jit, static_argnames=("n_out", "edge_split"))
def _segsum(xA, xB, srcp, dstp, n_out, edge_split=False):
    e_pad = srcp.shape[0]
    h = xA.shape[1]
    n_pad = _rup(n_out + 8, NS * GL)
    ept = e_pad // (NC * NS if edge_split else NS)
    nstep = ept // (NG * GL)
    total_groups = nstep * NG
    zspan = n_pad // NS
    nz = zspan // GL

    @functools.partial(
        pl.kernel,
        out_type=jax.ShapeDtypeStruct((NC, n_pad, h), jnp.float32),
        mesh=_mesh(),
        scratch_types=[
            [pltpu.VMEM((GL,), jnp.int32)] * NG,
            [pltpu.VMEM((GL,), jnp.int32)] * NG,
            [pltpu.VMEM((GL, h), jnp.float32)] * NG,
            pltpu.VMEM_SHARED((n_pad, h), jnp.float32),
            pltpu.SemaphoreType.DMA,
            pltpu.SemaphoreType.DMA,
            pltpu.SemaphoreType.DMA,
        ],
        compiler_params=_SC_PARAMS,
    )
    def k(xA_h, xB_h, src_h, dst_h, out_h, sbufs, dbufs, rowss, acc, sem_i, sem_g, sem_s):
        c = lax.axis_index("c")
        s = lax.axis_index("s")
        base = s * zspan
        zv = jnp.zeros((16,), jnp.float32)

        def zb(i, carry):
            for j in range(h // 16):
                rowss[0][i, pl.ds(j * 16, 16)] = zv
            return carry

        lax.fori_loop(0, GL, zb, 0)

        def zc(i, carry):
            pltpu.sync_copy(rowss[0], acc.at[pl.ds(base + i * GL, GL)])
            return carry

        lax.fori_loop(0, nz, zc, 0)
        plsc.subcore_barrier()

        if edge_split:
            ebase = (c * NS + s) * ept
        else:
            ebase = s * ept

        def load_group(m, grp):
            off = ebase + m * GL
            pltpu.async_copy(src_h.at[pl.ds(off, GL)], sbufs[grp], sem_i)
            pltpu.async_copy(dst_h.at[pl.ds(off, GL)], dbufs[grp], sem_i)

        def wait_idx():
            pltpu.make_async_copy(src_h.at[pl.ds(0, GL)], sbufs[0], sem_i).wait()
            pltpu.make_async_copy(dst_h.at[pl.ds(0, GL)], dbufs[0], sem_i).wait()

        def fire_gather(grp):
            if edge_split:
                pltpu.async_copy(xA_h.at[sbufs[grp]], rowss[grp], sem_g)
            else:
                @pl.when(c == 0)
                def _():
                    pltpu.async_copy(xA_h.at[sbufs[grp]], rowss[grp], sem_g)

                @pl.when(c == 1)
                def _():
                    pltpu.async_copy(xB_h.at[sbufs[grp]], rowss[grp], sem_g)

        def wait_gather(grp):
            pltpu.make_async_copy(xA_h.at[sbufs[grp]], rowss[grp], sem_g).wait()

        def fire_scatter(grp):
            pltpu.async_copy(rowss[grp], acc.at[dbufs[grp]], sem_s, add=True)

        def wait_scatter(grp):
            pltpu.make_async_copy(rowss[grp], acc.at[dbufs[grp]], sem_s).wait()

        load_group(0, 0)
        wait_idx()
        fire_gather(0)

        def body(m, carry):
            for grp in range(NG):
                g_idx = m * NG + grp
                nxt = (grp + 1) % NG

                @pl.when(g_idx + 1 < total_groups)
                def _():
                    load_group(g_idx + 1, nxt)
                    wait_idx()
                    fire_gather(nxt)

                wait_gather(grp)
                fire_scatter(grp)
                wait_scatter(grp)
            return carry

        lax.fori_loop(0, nstep, body, 0)
        plsc.subcore_barrier()

        def dr(i, carry):
            sl = pl.ds(base + i * GL, GL)
            pltpu.sync_copy(acc.at[sl], out_h.at[c, sl])
            return carry

        lax.fori_loop(0, nz, dr, 0)

    out = k(xA, xB, srcp, dstp)
    if edge_split:
        return out[0, :n_out] + out[1, :n_out]
    return jnp.concatenate([out[0, :n_out], out[1, :n_out]], axis=1)


@functools.partial(jax.jit, static_argnames=("n_out",))
def _take_rows(tA, tB, idxp, n_out):
    h = tA.shape[1]
    n_outp = idxp.shape[0]
    span = n_outp // NS
    nstep = span // (NG * GL)
    total_groups = nstep * NG

    @functools.partial(
        pl.kernel,
        out_type=jax.ShapeDtypeStruct((NC, n_outp, h), jnp.float32),
        mesh=_mesh(),
        scratch_types=[
            [pltpu.VMEM((GL,), jnp.int32)] * NG,
            [pltpu.VMEM((GL, h), jnp.float32)] * NG,
            pltpu.SemaphoreType.DMA,
            pltpu.SemaphoreType.DMA,
            pltpu.SemaphoreType.DMA,
        ],
        compiler_params=_SC_PARAMS,
    )
    def k(tA_h, tB_h, idx_h, out_h, ibufs, rowss, sem_i, sem_g, sem_o):
        c = lax.axis_index("c")
        s = lax.axis_index("s")
        ebase = s * span

        def load_group(m, grp):
            pltpu.async_copy(idx_h.at[pl.ds(ebase + m * GL, GL)], ibufs[grp], sem_i)

        def wait_idx():
            pltpu.make_async_copy(idx_h.at[pl.ds(0, GL)], ibufs[0], sem_i).wait()

        def fire_gather(grp):
            @pl.when(c == 0)
            def _():
                pltpu.async_copy(tA_h.at[ibufs[grp]], rowss[grp], sem_g)

            @pl.when(c == 1)
            def _():
                pltpu.async_copy(tB_h.at[ibufs[grp]], rowss[grp], sem_g)

        def wait_gather(grp):
            pltpu.make_async_copy(tA_h.at[ibufs[grp]], rowss[grp], sem_g).wait()

        def fire_out(m, grp):
            pltpu.async_copy(rowss[grp], out_h.at[c, pl.ds(ebase + m * GL, GL)], sem_o)

        def wait_out(m, grp):
            pltpu.make_async_copy(rowss[grp], out_h.at[c, pl.ds(ebase + m * GL, GL)], sem_o).wait()

        load_group(0, 0)
        wait_idx()
        fire_gather(0)

        def body(m, carry):
            for grp in range(NG):
                g_idx = m * NG + grp
                nxt = (grp + 1) % NG

                @pl.when(g_idx + 1 < total_groups)
                def _():
                    load_group(g_idx + 1, nxt)
                    wait_idx()
                    fire_gather(nxt)

                wait_gather(grp)
                fire_out(g_idx, grp)
                wait_out(g_idx, grp)
            return carry

        lax.fori_loop(0, nstep, body, 0)

    out = k(tA, tB, idxp)
    return jnp.concatenate([out[0, :n_out], out[1, :n_out]], axis=1)


def _pad_edges(src, dst, n_out, nworkers):
    unit = nworkers * NG * GL
    e = src.shape[0]
    e_pad = _rup(e, unit)
    pad = e_pad - e
    srcp = jnp.concatenate([src, jnp.zeros((pad,), jnp.int32)])
    dstp = jnp.concatenate([dst, jnp.full((pad,), n_out, jnp.int32)])
    return srcp, dstp


def _pad_idx(idx):
    n = idx.shape[0]
    n_pad = _rup(n, NS * NG * GL)
    return jnp.concatenate([idx, jnp.zeros((n_pad - n,), jnp.int32)])


_JNP_SEG = False


def _segsum_jnp(xA, xB, srcp, dstp, n_out):
    x = jnp.concatenate([xA, xB], axis=1)
    return jax.ops.segment_sum(jnp.take(x, srcp, axis=0), dstp, num_segments=n_out + 1)[:n_out]


def _take_jnp(tA, tB, idxp, n_out):
    t = jnp.concatenate([tA, tB], axis=1)
    return jnp.take(t, idxp[:n_out], axis=0)


def _segsum_jnp_full(h, srcp, dstp, n_out):
    return jax.ops.segment_sum(jnp.take(h, srcp, axis=0), dstp, num_segments=n_out + 1)[:n_out]


def _halves(x):
    hh = x.shape[1] // 2
    return x[:, :hh], x[:, hh:]


def _bn(x, g, be):
    m = jnp.mean(x, axis=0)
    v = jnp.var(x, axis=0)
    return (x - m) * lax.rsqrt(v + 1e-5) * g + be


def _spconv(x, srcp, dstp, Ws, Wn, b, n):
    hn = x @ Wn
    if _JNP_SEG:
        agg = jax.ops.segment_sum(jnp.take(hn, srcp, axis=0), dstp, num_segments=n + 1)[:n]
    else:
        hnA, hnB = _halves(hn)
        agg = _segsum(hnA, hnB, srcp, dstp, n)
    return x @ Ws + agg + b


def _res(x, srcp, dstp, p, nm, n):
    h = jax.nn.relu(_bn(_spconv(x, srcp, dstp, p[nm + "_c1_Ws"], p[nm + "_c1_Wn"], p[nm + "_c1_b"], n),
                        p[nm + "_bn1_g"], p[nm + "_bn1_be"]))
    h = _bn(_spconv(h, srcp, dstp, p[nm + "_c2_Ws"], p[nm + "_c2_Wn"], p[nm + "_c2_b"], n),
            p[nm + "_bn2_g"], p[nm + "_bn2_be"])
    return jax.nn.relu(h + x)


def kernel(x, params, edge_index0, edge_index1, edge_index2, pool_idx1, pool_idx2):
    p = params
    s0p, d0p = _pad_edges(edge_index0[0], edge_index0[1], N0, NS)
    s1p, d1p = _pad_edges(edge_index1[0], edge_index1[1], N1, NS)
    s2p, d2p = _pad_edges(edge_index2[0], edge_index2[1], N2, NS)
    ar0 = jnp.arange(N0, dtype=jnp.int32)
    ar1 = jnp.arange(N1, dtype=jnp.int32)
    pool1s, pool1d = _pad_edges(ar0, pool_idx1.astype(jnp.int32), N1, NS)
    pool2s, pool2d = _pad_edges(ar1, pool_idx2.astype(jnp.int32), N2, NS)

    z = _spconv(x, s0p, d0p, p["stem_c_Ws"], p["stem_c_Wn"], p["stem_c_b"], N0)
    x0 = jax.nn.relu(_bn(z, p["stem_bn_g"], p["stem_bn_be"]))

    x1 = _res(x0, s0p, d0p, p, "enc1", N0)

    _tak = _take_jnp if _JNP_SEG else _take_rows
    if _JNP_SEG:
        c1 = _segsum_jnp_full(x1 @ p["down1_W"], pool1s, pool1d, N1) + p["down1_b"]
    else:
        hpA, hpB = _halves(x1 @ p["down1_W"])
        c1 = _segsum(hpA, hpB, pool1s, pool1d, N1) + p["down1_b"]

    x2 = _res(c1, s1p, d1p, p, "enc2", N1)

    if _JNP_SEG:
        c2 = _segsum_jnp_full(x2 @ p["down2_W"], pool2s, pool2d, N2) + p["down2_b"]
    else:
        hqA, hqB = _halves(x2 @ p["down2_W"])
        c2 = _segsum(hqA, hqB, pool2s, pool2d, N2) + p["down2_b"]

    x3 = _res(c2, s2p, d2p, p, "bott", N2)

    if _JNP_SEG:
        u1 = jnp.take(x3 @ p["up1_W"], pool_idx2, axis=0) + p["up1_b"]
    else:
        t1A, t1B = _halves(x3 @ p["up1_W"] + p["up1_b"])
        u1 = _tak(t1A, t1B, _pad_idx(pool_idx2.astype(jnp.int32)), N1)

    y1 = _res(u1, s1p, d1p, p, "dec1", N1)

    if _JNP_SEG:
        u2 = jnp.take(y1 @ p["up2_W"], pool_idx1, axis=0) + p["up2_b"]
    else:
        t2A, t2B = _halves(y1 @ p["up2_W"] + p["up2_b"])
        u2 = _tak(t2A, t2B, _pad_idx(pool_idx1.astype(jnp.int32)), N0)

    y0 = _res(u2, s0p, d0p, p, "dec2", N0)
    return y0 @ p["head_W"] + p["head_b"]
